# Initial kernel scaffold; baseline (speedup 1.0000x reference)
import jax
import jax.numpy as jnp
from jax.experimental import pallas as pl


def kernel(x, emb, fc, bias, W1, b1, W2, b2, W3, b3):
    raise NotImplementedError("write your pallas kernel here")


# speedup vs baseline: 1.3172x; 1.3172x over previous
import functools

import jax
import jax.numpy as jnp
import numpy as np
from jax import lax
from jax.experimental import pallas as pl
from jax.experimental.pallas import tpu as pltpu
from jax.experimental.pallas import tpu_sc as plsc

_FIELD_DIMS = [38462] * 26
_NUM_FIELDS = 26
_VOCAB = sum(_FIELD_DIMS)
_EMBED_DIM = 16
_BATCH = 16384
_MLP_IN = _NUM_FIELDS * _EMBED_DIM
_OFFSETS = np.concatenate(([0], np.cumsum(_FIELD_DIMS)[:-1])).astype(np.int32)

_NW = 32
_N_IDX = _BATCH * _NUM_FIELDS
_PER_W = _N_IDX // _NW
_CHUNK = 3328
_NCHUNK = _PER_W // _CHUNK


def _sc_gather(emb, fc1, idx):
    mesh = plsc.VectorSubcoreMesh(core_axis_name="c", subcore_axis_name="s")

    @functools.partial(
        pl.kernel,
        out_type=(
            jax.ShapeDtypeStruct((_N_IDX, _EMBED_DIM), jnp.float32),
            jax.ShapeDtypeStruct((_N_IDX,), jnp.float32),
        ),
        mesh=mesh,
        scratch_types=[
            pltpu.VMEM((_CHUNK,), jnp.int32),
            pltpu.VMEM((_CHUNK, _EMBED_DIM), jnp.float32),
            pltpu.VMEM((_CHUNK,), jnp.float32),
            pltpu.SemaphoreType.DMA,
            pltpu.SemaphoreType.DMA,
        ],
        compiler_params=pltpu.CompilerParams(use_tc_tiling_on_sc=False),
    )
    def k(emb_hbm, fc_hbm, idx_hbm, out_hbm, fcg_hbm, idx_v, rows_v, fcr_v, s1, s2):
        wid = lax.axis_index("s") * 2 + lax.axis_index("c")
        base = wid * _PER_W

        def body(t, carry):
            st = base + t * _CHUNK
            pltpu.sync_copy(idx_hbm.at[pl.ds(st, _CHUNK)], idx_v)
            c1 = pltpu.async_copy(emb_hbm.at[idx_v], rows_v, s1)
            c2 = pltpu.async_copy(fc_hbm.at[idx_v], fcr_v, s2)
            c1.wait()
            c2.wait()
            pltpu.sync_copy(rows_v, out_hbm.at[pl.ds(st, _CHUNK)])
            pltpu.sync_copy(fcr_v, fcg_hbm.at[pl.ds(st, _CHUNK)])
            return carry

        lax.fori_loop(0, _NCHUNK, body, 0)

    return k(emb, fc1, idx)


def _tc_body(ex_ref, fcg_ref, s_ref, w1_ref, b1_ref, w2_ref, b2_ref, w3_ref,
             cb_ref, out_ref):
    ex = ex_ref[...]
    rowsum = jnp.dot(ex, s_ref[...], preferred_element_type=jnp.float32)
    fm = 0.5 * (jnp.sum(rowsum * rowsum, axis=1) - jnp.sum(ex * ex, axis=1))
    lin = jnp.sum(fcg_ref[...], axis=1)
    h1 = jnp.maximum(
        jnp.dot(ex, w1_ref[...], preferred_element_type=jnp.float32)
        + b1_ref[...], 0.0)
    h2 = jnp.maximum(
        jnp.dot(h1, w2_ref[...], preferred_element_type=jnp.float32)
        + b2_ref[...], 0.0)
    mlp = jnp.sum(h2 * w3_ref[...], axis=1)
    out_ref[...] = lin + fm + mlp + cb_ref[0, 0]


def _tc_compute(ex, fcg, W1, b1, W2, b2, W3, cb, block=1024):
    sel = np.zeros((_MLP_IN, _EMBED_DIM), np.float32)
    for f in range(_NUM_FIELDS):
        for d in range(_EMBED_DIM):
            sel[f * _EMBED_DIM + d, d] = 1.0
    sel = jnp.asarray(sel)
    grid = (_BATCH // block,)
    return pl.pallas_call(
        _tc_body,
        grid=grid,
        in_specs=[
            pl.BlockSpec((block, _MLP_IN), lambda i: (i, 0)),
            pl.BlockSpec((block, _NUM_FIELDS), lambda i: (i, 0)),
            pl.BlockSpec((_MLP_IN, _EMBED_DIM), lambda i: (0, 0)),
            pl.BlockSpec((_MLP_IN, 128), lambda i: (0, 0)),
            pl.BlockSpec((1, 128), lambda i: (0, 0)),
            pl.BlockSpec((128, 64), lambda i: (0, 0)),
            pl.BlockSpec((1, 64), lambda i: (0, 0)),
            pl.BlockSpec((1, 64), lambda i: (0, 0)),
            pl.BlockSpec((1, 1), lambda i: (0, 0)),
        ],
        out_specs=pl.BlockSpec((block,), lambda i: (i,)),
        out_shape=jax.ShapeDtypeStruct((_BATCH,), jnp.float32),
    )(ex, fcg, sel, W1, b1, W2, b2, W3, cb)


def kernel(x, emb, fc, bias, W1, b1, W2, b2, W3, b3):
    idx = (x.astype(jnp.int32)
           + jnp.asarray(_OFFSETS, jnp.int32)[None, :]).reshape(-1)
    ex_flat, fcg_flat = _sc_gather(emb, fc.reshape(-1), idx)
    ex = ex_flat.reshape(_BATCH, _MLP_IN)
    fcg = fcg_flat.reshape(_BATCH, _NUM_FIELDS)
    cb = (bias + b3).reshape(1, 1)
    return _tc_compute(ex, fcg, W1.astype(jnp.float32), b1.reshape(1, 128),
                       W2, b2.reshape(1, 64), W3.reshape(1, 64), cb)
